# baseline (speedup 1.0000x reference)
import numpy as np
import jax
import jax.numpy as jnp
from jax import lax
from jax.experimental import pallas as pl
from jax.experimental.pallas import tpu as pltpu

_N = 64
_E = _N * (_N - 1)
_NODE_CH = 1280
_ACT_CH = 4
_BBOX_POS_MAX = 1024.0

_OFF_B1 = 0
_OFF_B2 = 512
_OFF_B3 = 768
_OFF_NCIB = 2048
_OFF_EPB1 = 2176
_OFF_EPB2 = 2432
_OFF_EPB3 = 2560
_OFF_EPROCB1 = 2688
_OFF_GBIAS = 2816
_OFF_PRIROW = 2944
_SMALLS_W = 3072


def _edge_onehots():
    idx = np.arange(_N)
    ii, jj = np.meshgrid(idx, idx, indexing="ij")
    mask = ii != jj
    src = ii[mask]
    dst = jj[mask]
    S = np.zeros((_E, _N), np.float32)
    S[np.arange(_E), src] = 1.0
    D = np.zeros((_E, _N), np.float32)
    D[np.arange(_E), dst] = 1.0
    return S, D


def _fused(roihl, attrp, smalls,
           W1, W2, W3, W2g, epW2,
           epW1sd, epW3, eprocW1, Wtail,
           SD, Dt, Rsum,
           out_act, out_nce, out_ea):
    f32 = jnp.float32
    bf16 = jnp.bfloat16
    i32 = jnp.int32

    def _split(a):
        hi = a.astype(bf16)
        lo = (a - hi.astype(f32)).astype(bf16)
        return hi, lo

    def _d(u, v):
        return jnp.dot(u, v, preferred_element_type=f32)

    def dot(a, b):
        ahi, alo = _split(a)
        bhi, blo = _split(b)
        return _d(ahi, bhi) + _d(ahi, blo) + _d(alo, bhi)

    def odot(sb, b):
        bhi, blo = _split(b)
        return _d(sb, bhi) + _d(sb, blo)

    sm = smalls[...]
    b1 = sm[:, _OFF_B1:_OFF_B1 + 512]
    b2 = sm[:, _OFF_B2:_OFF_B2 + 256]
    b3 = sm[:, _OFF_B3:_OFF_B3 + 1280]
    ncib = sm[:, _OFF_NCIB:_OFF_NCIB + 5]
    epb1 = sm[:, _OFF_EPB1:_OFF_EPB1 + 256]
    epb2 = sm[:, _OFF_EPB2:_OFF_EPB2 + 64]
    epb3 = sm[:, _OFF_EPB3:_OFF_EPB3 + 3]
    eprocb1 = sm[:, _OFF_EPROCB1:_OFF_EPROCB1 + 128]
    gbias = sm[:, _OFF_GBIAS:_OFF_GBIAS + 4]
    pri_row = sm[:, _OFF_PRIROW:_OFF_PRIROW + 64]

    one = f32(1.0)
    zero = f32(0.0)
    SDb = SD[...]
    Sb = SDb[:, :64]
    Df = SDb[:, 64:].astype(f32)
    Dtb = Dt[...]
    Rsumb = Rsum[...]

    rhi = roihl[:, :512]
    rlo = roihl[:, 512:]
    W1hi, W1lo = _split(W1[...])
    h = jax.nn.relu(_d(rhi, W1hi) + _d(rhi, W1lo) + _d(rlo, W1hi) + b1)
    h = jax.nn.relu(dot(h, W2[...]) + b2)
    x = dot(h, W3[...]) + b3

    Gall = dot(x, W2g[...])
    XCt = dot(x, Wtail[...])
    out_nce[...] = jax.nn.sigmoid(XCt[:, 0:5] + ncib)
    rootterm = XCt[:, 5:9]
    bterm = XCt[:, 9:13]

    attr = attrp[:, 0:8]
    pri_col = attrp[:, 8:9]
    AsAd = dot(attr, epW1sd[...])
    AB = jnp.concatenate([AsAd[:, :256], AsAd[:, 256:]], axis=0)
    e1 = jax.nn.relu(odot(SDb, AB) + epb1)
    e2 = jax.nn.relu(dot(e1, epW2[...]) + epb2)
    ea3 = jax.nn.sigmoid(dot(e2, epW3[...]) + epb3)
    pdd = pri_col - pri_row
    Pd = jnp.where(pdd > 0.0, one, zero).astype(bf16)
    SPd = _d(Sb, Pd)
    hp = jnp.sum(SPd * Df, axis=1, keepdims=True)
    ea4 = jnp.concatenate([ea3, hp], axis=1)
    out_ea[...] = ea4

    H = jax.nn.relu(dot(ea4, eprocW1[...]) + eprocb1)
    SG = odot(Sb, Gall)
    Ht = jnp.concatenate([H, H, H, H], axis=1)
    Phi, Plo = _split(Ht * SG)
    T = _d(Dtb, Phi) + _d(Dtb, Plo)
    Thi, Tlo = _split(T)
    aggr1 = _d(Thi, Rsumb) + _d(Tlo, Rsumb)
    tot = jnp.sum(bterm, axis=0, keepdims=True)
    aggr = aggr1 + (tot - bterm)
    out_act[...] = aggr + rootterm + gbias


def kernel(roi_features, batch_bboxes, batch_directions, batch_priorities,
           ncp_W1, ncp_b1, ncp_W2, ncp_b2, ncp_W3, ncp_b3,
           nci_W, nci_b,
           ep_W1, ep_b1, ep_W2, ep_b2, ep_W3, ep_b3,
           eproc_W1, eproc_b1, eproc_W2, eproc_b2,
           gnn_root, gnn_bias):
    f32 = jnp.float32
    bf16 = jnp.bfloat16
    roi0 = roi_features[0]
    rhi = roi0.astype(bf16)
    rlo = (roi0 - rhi.astype(f32)).astype(bf16)
    roihl = jnp.concatenate([rhi, rlo], axis=1)

    attrp = jnp.concatenate(
        [batch_bboxes[0] / _BBOX_POS_MAX, batch_directions[0],
         batch_priorities[0][:, None]], axis=-1)

    def pad_to(v, width):
        v = v.reshape(-1).astype(f32)
        return jnp.pad(v, (0, width - v.shape[0]))

    smalls = jnp.concatenate([
        pad_to(ncp_b1, 512),
        pad_to(ncp_b2, 256),
        pad_to(ncp_b3, 1280),
        pad_to(nci_b, 128),
        pad_to(ep_b1, 256),
        pad_to(ep_b2, 128),
        pad_to(ep_b3, 128),
        pad_to(eproc_b1, 128),
        pad_to(gnn_bias, 128),
        pad_to(batch_priorities[0], 128),
    ]).reshape(1, _SMALLS_W)

    W2g = eproc_W2.T.reshape(_NODE_CH, _ACT_CH * 128)
    b2r = eproc_b2.reshape(_NODE_CH, _ACT_CH)
    Wtail = jnp.concatenate([nci_W, gnn_root, b2r], axis=1)
    epW1sd = jnp.concatenate([ep_W1[:8], ep_W1[8:]], axis=1)

    S_np, D_np = _edge_onehots()
    SD = jnp.asarray(np.concatenate([S_np, D_np], axis=1), dtype=bf16)
    Dt = jnp.asarray(D_np.T, dtype=bf16)
    Rsum_np = np.zeros((_ACT_CH * 128, _ACT_CH), np.float32)
    for o in range(_ACT_CH):
        Rsum_np[o * 128:(o + 1) * 128, o] = 1.0
    Rsum = jnp.asarray(Rsum_np, dtype=bf16)

    out_shape = (
        jax.ShapeDtypeStruct((_N, _ACT_CH), f32),
        jax.ShapeDtypeStruct((_N, 5), f32),
        jax.ShapeDtypeStruct((_E, 4), f32),
    )
    return pl.pallas_call(
        _fused,
        out_shape=out_shape,
    )(roihl, attrp, smalls,
      ncp_W1, ncp_W2, ncp_W3, W2g, ep_W2,
      epW1sd, ep_W3, eproc_W1, Wtail,
      SD, Dt, Rsum)


# speedup vs baseline: 1.1245x; 1.1245x over previous
import numpy as np
import jax
import jax.numpy as jnp
from jax import lax
from jax.experimental import pallas as pl
from jax.experimental.pallas import tpu as pltpu

_N = 64
_E = _N * (_N - 1)
_NODE_CH = 1280
_ACT_CH = 4
_BBOX_POS_MAX = 1024.0


def _edge_onehots():
    idx = np.arange(_N)
    ii, jj = np.meshgrid(idx, idx, indexing="ij")
    mask = ii != jj
    src = ii[mask]
    dst = jj[mask]
    S = np.zeros((_E, _N), np.float32)
    S[np.arange(_E), src] = 1.0
    D = np.zeros((_E, _N), np.float32)
    D[np.arange(_E), dst] = 1.0
    return S, D


def _fused(roi, attr, pri_col, pri_row,
           W1, b1, W2, b2, W3, b3,
           ncib,
           epW1sd, epb1, epW2, epb2, epW3, epb3,
           eprocW1, eprocb1, W2g, Wtail,
           gbias,
           out_act, out_nce, out_ea):
    f32 = jnp.float32
    bf16 = jnp.bfloat16

    def _split(a):
        hi = a.astype(bf16)
        lo = (a - hi.astype(f32)).astype(bf16)
        return hi, lo

    def _d(u, v):
        return jnp.dot(u, v, preferred_element_type=f32)

    def dot(a, b):
        ahi, alo = _split(a)
        bhi, blo = _split(b)
        return _d(ahi, bhi) + _d(ahi, blo) + _d(alo, bhi)

    def odot(sb, b):
        bhi, blo = _split(b)
        return _d(sb, bhi) + _d(sb, blo)

    def odot_r(a, sb):
        ahi, alo = _split(a)
        return _d(ahi, sb) + _d(alo, sb)

    i32 = jnp.int32
    one = f32(1.0)
    zero = f32(0.0)
    inv63 = f32(1.0 / 63.0)
    r = lax.broadcasted_iota(i32, (_E, 128), 0)
    c = lax.broadcasted_iota(i32, (_E, 128), 1)
    src = jnp.floor((r.astype(f32) + 0.5) * inv63).astype(i32)
    j0 = r - src * 63
    dstv = j0 + jnp.where(j0 >= src, 1, 0).astype(i32)
    SDf = (jnp.where(c == src, one, zero)
           + jnp.where(c == dstv + 64, one, zero))
    SDb = SDf.astype(bf16)
    Sb = SDb[:, :64]
    Df = SDf[:, 64:]
    r2 = lax.broadcasted_iota(i32, (_N, _E), 0)
    c2 = lax.broadcasted_iota(i32, (_N, _E), 1)
    src2 = jnp.floor((c2.astype(f32) + 0.5) * inv63).astype(i32)
    j02 = c2 - src2 * 63
    dst2 = j02 + jnp.where(j02 >= src2, 1, 0).astype(i32)
    Dtb = jnp.where(r2 == dst2, one, zero).astype(bf16)
    rr = lax.broadcasted_iota(i32, (_ACT_CH * 128, _ACT_CH), 0)
    cc = lax.broadcasted_iota(i32, (_ACT_CH * 128, _ACT_CH), 1)
    rrg = jnp.floor((rr.astype(f32) + 0.5) * f32(1.0 / 128.0)).astype(i32)
    Rsumb = jnp.where(rrg == cc, one, zero).astype(bf16)

    h = jax.nn.relu(dot(roi[...], W1[...]) + b1[...])
    h = jax.nn.relu(dot(h, W2[...]) + b2[...])
    x = dot(h, W3[...]) + b3[...]

    Gall = dot(x, W2g[...])
    XCt = dot(x, Wtail[...])
    out_nce[...] = jax.nn.sigmoid(XCt[:, 0:5] + ncib[...])
    rootterm = XCt[:, 5:9]
    bterm = XCt[:, 9:13]

    AsAd = dot(attr[...], epW1sd[...])
    AB = jnp.concatenate([AsAd[:, :256], AsAd[:, 256:]], axis=0)
    e1 = jax.nn.relu(odot(SDb, AB) + epb1[...])
    e2 = jax.nn.relu(dot(e1, epW2[...]) + epb2[...])
    ea3 = jax.nn.sigmoid(dot(e2, epW3[...]) + epb3[...])
    pdd = pri_col[...] - pri_row[...]
    Pd = jnp.where(pdd > 0.0, f32(1.0), f32(0.0)).astype(bf16)
    SPd = _d(Sb, Pd)
    hp = jnp.sum(SPd * Df, axis=1, keepdims=True)
    ea4 = jnp.concatenate([ea3, hp], axis=1)
    out_ea[...] = ea4

    H = jax.nn.relu(dot(ea4, eprocW1[...]) + eprocb1[...])
    SG = odot(Sb, Gall)
    Ht = jnp.concatenate([H, H, H, H], axis=1)
    T = odot(Dtb, Ht * SG)
    aggr1 = odot_r(T, Rsumb)
    tot = jnp.sum(bterm, axis=0, keepdims=True)
    aggr = aggr1 + (tot - bterm)
    out_act[...] = aggr + rootterm + gbias[...]


def kernel(roi_features, batch_bboxes, batch_directions, batch_priorities,
           ncp_W1, ncp_b1, ncp_W2, ncp_b2, ncp_W3, ncp_b3,
           nci_W, nci_b,
           ep_W1, ep_b1, ep_W2, ep_b2, ep_W3, ep_b3,
           eproc_W1, eproc_b1, eproc_W2, eproc_b2,
           gnn_root, gnn_bias):
    f32 = jnp.float32
    roi0 = roi_features[0]
    attr0 = jnp.concatenate(
        [batch_bboxes[0] / _BBOX_POS_MAX, batch_directions[0]], axis=-1)
    pri0 = batch_priorities[0][:, None]

    pri_row = batch_priorities[0][None, :]

    W2g = eproc_W2.T.reshape(_NODE_CH, _ACT_CH * 128)
    b2r = eproc_b2.reshape(_NODE_CH, _ACT_CH)
    Wtail = jnp.concatenate([nci_W, gnn_root, b2r], axis=1)
    epW1sd = jnp.concatenate([ep_W1[:8], ep_W1[8:]], axis=1)

    row = lambda v: v.reshape(1, -1).astype(f32)

    out_shape = (
        jax.ShapeDtypeStruct((_N, _ACT_CH), f32),
        jax.ShapeDtypeStruct((_N, 5), f32),
        jax.ShapeDtypeStruct((_E, 4), f32),
    )
    return pl.pallas_call(
        _fused,
        out_shape=out_shape,
    )(roi0, attr0, pri0, pri_row,
      ncp_W1, row(ncp_b1), ncp_W2, row(ncp_b2), ncp_W3, row(ncp_b3),
      row(nci_b),
      epW1sd, row(ep_b1), ep_W2, row(ep_b2), ep_W3, row(ep_b3),
      eproc_W1, row(eproc_b1), W2g, Wtail,
      row(gnn_bias))
